# baseline (speedup 1.0000x reference)
import jax
import jax.numpy as jnp
from jax import lax
from jax.experimental import pallas as pl
from jax.experimental.pallas import tpu as pltpu
from jax.experimental.pallas import tpu_sc as plsc

B = 16384
SEQ = 20
NC = 2
NS = 16
NW = NC * NS
BPW = B // NW
TCH = 32
NTCH = BPW // TCH
L = 16


def _body(tidx_hbm, ttok_hbm, gtok_hbm, ttab_hbm, xtab_hbm, gtab_hbm,
          out_hbm,
          tidx_v, ttok_v, gtok_v, tbuf, gbuf, trowbuf, res80,
          inv_t, n0_t, inv_g, n0_g, ttab0, gtab0, gsem, wsem):
    wid = lax.axis_index("s") * NC + lax.axis_index("c")
    base = pl.multiple_of(wid * BPW, BPW)

    pltpu.sync_copy(tidx_hbm.at[pl.ds(base, BPW)], tidx_v)
    pltpu.sync_copy(ttok_hbm.at[:, pl.ds(base, BPW)], ttok_v)
    pltpu.sync_copy(gtok_hbm.at[:, pl.ds(base, BPW)], gtok_v)
    pltpu.sync_copy(xtab_hbm.at[0], ttab0)
    pltpu.sync_copy(gtab_hbm.at[0], gtab0)

    def fire(c, b):
        off = pl.multiple_of(c * TCH, TCH)
        pltpu.async_copy(
            ttab_hbm.at[tidx_v.at[pl.ds(off, TCH)]],
            trowbuf.at[b], gsem.at[b])
        for p in range(SEQ):
            pltpu.async_copy(
                xtab_hbm.at[ttok_v.at[p, pl.ds(off, TCH)]],
                tbuf.at[b, p], gsem.at[b])
            pltpu.async_copy(
                gtab_hbm.at[gtok_v.at[p, pl.ds(off, TCH)]],
                gbuf.at[b, p], gsem.at[b])

    def drain(b):
        pltpu.make_async_copy(
            ttab_hbm.at[tidx_v.at[pl.ds(0, TCH)]],
            trowbuf.at[b], gsem.at[b]).wait()
        for p in range(SEQ):
            pltpu.make_async_copy(
                xtab_hbm.at[ttok_v.at[p, pl.ds(0, TCH)]],
                tbuf.at[b, p], gsem.at[b]).wait()
            pltpu.make_async_copy(
                gtab_hbm.at[gtok_v.at[p, pl.ds(0, TCH)]],
                gbuf.at[b, p], gsem.at[b]).wait()

    def cnt_body(i, _):
        off = pl.multiple_of(i * L, L)
        ct = jnp.zeros((L,), jnp.float32)
        cg = jnp.zeros((L,), jnp.float32)
        for p in range(SEQ):
            t = ttok_v[p, pl.ds(off, L)]
            g = gtok_v[p, pl.ds(off, L)]
            ct = ct + jnp.where(t != 0, 1.0, 0.0).astype(jnp.float32)
            cg = cg + jnp.where(g != 0, 1.0, 0.0).astype(jnp.float32)
        inv_t[pl.ds(off, L)] = 1.0 / jnp.maximum(ct, 1.0)
        n0_t[pl.ds(off, L)] = float(SEQ) - ct
        inv_g[pl.ds(off, L)] = 1.0 / jnp.maximum(cg, 1.0)
        n0_g[pl.ds(off, L)] = float(SEQ) - cg
        return 0

    def compute(c, b):
        off = pl.multiple_of(c * TCH, TCH)

        def grp_body(g, _):
            goff = pl.multiple_of(g * L, L)
            iv_tv = inv_t[pl.ds(pl.multiple_of(off + goff, L), L)]
            nv_tv = n0_t[pl.ds(pl.multiple_of(off + goff, L), L)]
            iv_gv = inv_g[pl.ds(pl.multiple_of(off + goff, L), L)]
            nv_gv = n0_g[pl.ds(pl.multiple_of(off + goff, L), L)]
            for j in range(L):
                r = goff + j
                iv_t = iv_tv[j]
                nv_t = nv_tv[j]
                iv_g = iv_gv[j]
                nv_g = nv_gv[j]
                for h in range(2):
                    s = tbuf[b, 0, r, pl.ds(h * L, L)]
                    for p in range(1, SEQ):
                        s = s + tbuf[b, p, r, pl.ds(h * L, L)]
                    t0 = ttab0[pl.ds(h * L, L)]
                    res80[b, r, pl.ds(32 + h * L, L)] = (s - nv_t * t0) * iv_t
                    res80[b, r, pl.ds(h * L, L)] = \
                        trowbuf[b, r, pl.ds(h * L, L)]
                s = gbuf[b, 0, r, :]
                for p in range(1, SEQ):
                    s = s + gbuf[b, p, r, :]
                res80[b, r, pl.ds(64, L)] = (s - nv_g * gtab0[:]) * iv_g
            return 0

        lax.fori_loop(0, TCH // L, grp_body, 0)
        pltpu.async_copy(
            res80.at[b],
            out_hbm.at[pl.ds(base + off, TCH)],
            wsem.at[b])

    def drain_res(b):
        pltpu.make_async_copy(res80.at[b], out_hbm.at[pl.ds(0, TCH)],
                              wsem.at[b]).wait()

    fire(0, 0)
    fire(1, 1)
    lax.fori_loop(0, BPW // L, cnt_body, 0)

    def pair(c0, _):
        for b in range(2):
            c = c0 + b
            drain(b)

            @pl.when(c >= 2)
            def _():
                drain_res(b)

            compute(c, b)

            @pl.when(c + 2 < NTCH)
            def _():
                fire(c + 2, b)
        return 0

    lax.fori_loop(0, NTCH // 2, lambda i, _: pair(i * 2, _), 0)
    drain_res(0)
    drain_res(1)


@jax.jit
def _run(tidx, ttok_t, gtok_t, ttab, xtab, gtab):
    mesh = plsc.VectorSubcoreMesh(
        core_axis_name="c", subcore_axis_name="s",
        num_cores=NC, num_subcores=NS)
    return pl.kernel(
        _body,
        out_type=jax.ShapeDtypeStruct((B, 80), jnp.float32),
        mesh=mesh,
        scratch_types=[
            pltpu.VMEM((BPW,), jnp.int32),
            pltpu.VMEM((SEQ, BPW), jnp.int32),
            pltpu.VMEM((SEQ, BPW), jnp.int32),
            pltpu.VMEM((2, SEQ, TCH, 32), jnp.float32),
            pltpu.VMEM((2, SEQ, TCH, 16), jnp.float32),
            pltpu.VMEM((2, TCH, 32), jnp.float32),
            pltpu.VMEM((2, TCH, 80), jnp.float32),
            pltpu.VMEM((BPW,), jnp.float32),
            pltpu.VMEM((BPW,), jnp.float32),
            pltpu.VMEM((BPW,), jnp.float32),
            pltpu.VMEM((BPW,), jnp.float32),
            pltpu.VMEM((32,), jnp.float32),
            pltpu.VMEM((16,), jnp.float32),
            pltpu.SemaphoreType.DMA((2,)),
            pltpu.SemaphoreType.DMA((2,)),
        ],
        compiler_params=pltpu.CompilerParams(use_tc_tiling_on_sc=False),
    )(tidx, ttok_t, gtok_t, ttab, xtab, gtab)


def kernel(movie_title_idx, title_tokens, genre_tokens,
           title_table, text_table, genre_table):
    tidx = movie_title_idx.astype(jnp.int32)
    ttok_t = title_tokens.astype(jnp.int32).T
    gtok_t = genre_tokens.astype(jnp.int32).T
    return _run(tidx, ttok_t, gtok_t, title_table, text_table, genre_table)


# speedup vs baseline: 1.6618x; 1.6618x over previous
import jax
import jax.numpy as jnp
from jax import lax
from jax.experimental import pallas as pl
from jax.experimental.pallas import tpu as pltpu
from jax.experimental.pallas import tpu_sc as plsc

B = 16384
SEQ = 20
NC = 2
NS = 16
NW = NC * NS
BPW = B // NW
TCH = 32
NTCH = BPW // TCH
LCH = 128
NLCH = BPW // LCH
L = 16


def _splat_idx(i):
    return jnp.full((L,), i, dtype=jnp.int32)


def _body(tidx_hbm, ttok_hbm, gtok_hbm, ttab_hbm, xtab_hbm, gtab_hbm,
          out_hbm,
          tidx_v, ttok_v, gtok_v, tbuf, gbuf, trows,
          res_t, res_g, inv_t, n0_t, inv_g, n0_g, ttab0, gtab0,
          sem, gsem, hsem, wsem, vsem):
    wid = lax.axis_index("s") * NC + lax.axis_index("c")
    base = pl.multiple_of(wid * BPW, BPW)

    pltpu.sync_copy(tidx_hbm.at[pl.ds(base, BPW)], tidx_v)
    pltpu.sync_copy(ttok_hbm.at[:, pl.ds(base, BPW)], ttok_v)
    pltpu.sync_copy(gtok_hbm.at[:, pl.ds(base, BPW)], gtok_v)
    pltpu.sync_copy(xtab_hbm.at[0], ttab0)
    pltpu.sync_copy(gtab_hbm.at[0], gtab0)

    for c in range(NLCH):
        pltpu.async_copy(
            ttab_hbm.at[tidx_v.at[pl.ds(c * LCH, LCH)]], trows, sem).wait()
        pltpu.sync_copy(trows,
                        out_hbm.at[pl.ds(base + c * LCH, LCH), pl.ds(0, 32)])

    def cnt_body(i, _):
        off = pl.multiple_of(i * L, L)
        ct = jnp.zeros((L,), jnp.float32)
        cg = jnp.zeros((L,), jnp.float32)
        for p in range(SEQ):
            t = ttok_v[p, pl.ds(off, L)]
            g = gtok_v[p, pl.ds(off, L)]
            ct = ct + jnp.where(t != 0, 1.0, 0.0).astype(jnp.float32)
            cg = cg + jnp.where(g != 0, 1.0, 0.0).astype(jnp.float32)
        inv_t[pl.ds(off, L)] = 1.0 / jnp.maximum(ct, 1.0)
        n0_t[pl.ds(off, L)] = float(SEQ) - ct
        inv_g[pl.ds(off, L)] = 1.0 / jnp.maximum(cg, 1.0)
        n0_g[pl.ds(off, L)] = float(SEQ) - cg
        return 0

    lax.fori_loop(0, BPW // L, cnt_body, 0)

    def fire_text(c, b):
        off = pl.multiple_of(c * TCH, TCH)
        for p in range(SEQ):
            pltpu.async_copy(
                xtab_hbm.at[ttok_v.at[p, pl.ds(off, TCH)]],
                tbuf.at[b, p], gsem.at[b])

    def drain_text(c, b):
        for p in range(SEQ):
            pltpu.make_async_copy(
                xtab_hbm.at[ttok_v.at[p, pl.ds(0, TCH)]],
                tbuf.at[b, p], gsem.at[b]).wait()

    def text_compute(c, b):
        off = pl.multiple_of(c * TCH, TCH)

        def grp_body(g, _):
            goff = pl.multiple_of(g * L, L)
            iv_vec = inv_t[pl.ds(pl.multiple_of(off + goff, L), L)]
            nv_vec = n0_t[pl.ds(pl.multiple_of(off + goff, L), L)]
            for j in range(L):
                r = goff + j
                iv = iv_vec[j]
                nv = nv_vec[j]
                for h in range(2):
                    s = tbuf[b, 0, r, pl.ds(h * L, L)]
                    for p in range(1, SEQ):
                        s = s + tbuf[b, p, r, pl.ds(h * L, L)]
                    t0 = ttab0[pl.ds(h * L, L)]
                    res_t[b, r, pl.ds(h * L, L)] = (s - nv * t0) * iv
            return 0

        lax.fori_loop(0, TCH // L, grp_body, 0)
        pltpu.async_copy(res_t.at[b],
                         out_hbm.at[pl.ds(base + off, TCH), pl.ds(32, 32)],
                         wsem.at[b])

    def drain_res_t(b):
        pltpu.make_async_copy(res_t.at[b],
                              out_hbm.at[pl.ds(0, TCH), pl.ds(32, 32)],
                              wsem.at[b]).wait()

    def fire_genre(c, b):
        off = pl.multiple_of(c * TCH, TCH)
        for p in range(SEQ):
            pltpu.async_copy(
                gtab_hbm.at[gtok_v.at[p, pl.ds(off, TCH)]],
                gbuf.at[b, p], hsem.at[b])

    def drain_genre(c, b):
        for p in range(SEQ):
            pltpu.make_async_copy(
                gtab_hbm.at[gtok_v.at[p, pl.ds(0, TCH)]],
                gbuf.at[b, p], hsem.at[b]).wait()

    def genre_compute(c, b):
        off = pl.multiple_of(c * TCH, TCH)

        def grp_body(g, _):
            goff = pl.multiple_of(g * L, L)
            iv_vec = inv_g[pl.ds(pl.multiple_of(off + goff, L), L)]
            nv_vec = n0_g[pl.ds(pl.multiple_of(off + goff, L), L)]
            for j in range(L):
                r = goff + j
                iv = iv_vec[j]
                nv = nv_vec[j]
                s = gbuf[b, 0, r, :]
                for p in range(1, SEQ):
                    s = s + gbuf[b, p, r, :]
                res_g[b, r, :] = (s - nv * gtab0[:]) * iv
            return 0

        lax.fori_loop(0, TCH // L, grp_body, 0)
        pltpu.async_copy(res_g.at[b],
                         out_hbm.at[pl.ds(base + off, TCH), pl.ds(64, 16)],
                         vsem.at[b])

    def drain_res_g(b):
        pltpu.make_async_copy(res_g.at[b],
                              out_hbm.at[pl.ds(0, TCH), pl.ds(64, 16)],
                              vsem.at[b]).wait()

    fire_text(0, 0)
    fire_text(1, 1)

    def text_pair(c0, _):
        for b in range(2):
            c = c0 + b
            drain_text(c, b)

            @pl.when(c >= 2)
            def _():
                drain_res_t(b)

            text_compute(c, b)

            @pl.when(c + 2 < NTCH)
            def _():
                fire_text(c + 2, b)
        return 0

    lax.fori_loop(0, NTCH // 2, lambda i, _: text_pair(i * 2, _), 0)
    drain_res_t(0)
    drain_res_t(1)

    fire_genre(0, 0)
    fire_genre(1, 1)

    def genre_pair(c0, _):
        for b in range(2):
            c = c0 + b
            drain_genre(c, b)

            @pl.when(c >= 2)
            def _():
                drain_res_g(b)

            genre_compute(c, b)

            @pl.when(c + 2 < NTCH)
            def _():
                fire_genre(c + 2, b)
        return 0

    lax.fori_loop(0, NTCH // 2, lambda i, _: genre_pair(i * 2, _), 0)
    drain_res_g(0)
    drain_res_g(1)


@jax.jit
def _run(tidx, ttok_t, gtok_t, ttab, xtab, gtab):
    mesh = plsc.VectorSubcoreMesh(
        core_axis_name="c", subcore_axis_name="s",
        num_cores=NC, num_subcores=NS)
    return pl.kernel(
        _body,
        out_type=jax.ShapeDtypeStruct((B, 80), jnp.float32),
        mesh=mesh,
        scratch_types=[
            pltpu.VMEM((BPW,), jnp.int32),
            pltpu.VMEM((SEQ, BPW), jnp.int32),
            pltpu.VMEM((SEQ, BPW), jnp.int32),
            pltpu.VMEM((2, SEQ, TCH, 32), jnp.float32),
            pltpu.VMEM((2, SEQ, TCH, 16), jnp.float32),
            pltpu.VMEM((LCH, 32), jnp.float32),
            pltpu.VMEM((2, TCH, 32), jnp.float32),
            pltpu.VMEM((2, TCH, 16), jnp.float32),
            pltpu.VMEM((BPW,), jnp.float32),
            pltpu.VMEM((BPW,), jnp.float32),
            pltpu.VMEM((BPW,), jnp.float32),
            pltpu.VMEM((BPW,), jnp.float32),
            pltpu.VMEM((32,), jnp.float32),
            pltpu.VMEM((16,), jnp.float32),
            pltpu.SemaphoreType.DMA,
            pltpu.SemaphoreType.DMA((2,)),
            pltpu.SemaphoreType.DMA((2,)),
            pltpu.SemaphoreType.DMA((2,)),
            pltpu.SemaphoreType.DMA((2,)),
        ],
        compiler_params=pltpu.CompilerParams(use_tc_tiling_on_sc=False),
    )(tidx, ttok_t, gtok_t, ttab, xtab, gtab)


def kernel(movie_title_idx, title_tokens, genre_tokens,
           title_table, text_table, genre_table):
    tidx = movie_title_idx.astype(jnp.int32)
    ttok_t = title_tokens.astype(jnp.int32).T
    gtok_t = genre_tokens.astype(jnp.int32).T
    return _run(tidx, ttok_t, gtok_t, title_table, text_table, genre_table)


# speedup vs baseline: 1.6904x; 1.0172x over previous
import jax
import jax.numpy as jnp
from jax import lax
from jax.experimental import pallas as pl
from jax.experimental.pallas import tpu as pltpu
from jax.experimental.pallas import tpu_sc as plsc

B = 16384
SEQ = 20
NC = 2
NS = 16
NW = NC * NS
BPW = B // NW
TCH = 32
NTCH = BPW // TCH
LCH = 128
NLCH = BPW // LCH
L = 16


def _splat_idx(i):
    return jnp.full((L,), i, dtype=jnp.int32)


def _body(tidx_hbm, ttok_hbm, gtok_hbm, ttab_hbm, xtab_hbm, gtab_hbm,
          out_hbm,
          tidx_v, ttok_v, gtok_v, tbuf, gbuf, trows,
          res_t, res_g, inv_t, n0_t, inv_g, n0_g, ttab0, gtab0,
          sem, gsem, hsem, wsem, vsem):
    wid = lax.axis_index("s") * NC + lax.axis_index("c")
    base = pl.multiple_of(wid * BPW, BPW)

    pltpu.sync_copy(tidx_hbm.at[pl.ds(base, BPW)], tidx_v)
    pltpu.sync_copy(ttok_hbm.at[:, pl.ds(base, BPW)], ttok_v)
    pltpu.sync_copy(gtok_hbm.at[:, pl.ds(base, BPW)], gtok_v)
    pltpu.sync_copy(xtab_hbm.at[0], ttab0)
    pltpu.sync_copy(gtab_hbm.at[0], gtab0)

    tcps = []
    for c in range(NLCH):
        tcps.append(pltpu.async_copy(
            ttab_hbm.at[tidx_v.at[pl.ds(c * LCH, LCH)]], trows.at[c], sem))

    def cnt_body(i, _):
        off = pl.multiple_of(i * L, L)
        ct = jnp.zeros((L,), jnp.float32)
        cg = jnp.zeros((L,), jnp.float32)
        for p in range(SEQ):
            t = ttok_v[p, pl.ds(off, L)]
            g = gtok_v[p, pl.ds(off, L)]
            ct = ct + jnp.where(t != 0, 1.0, 0.0).astype(jnp.float32)
            cg = cg + jnp.where(g != 0, 1.0, 0.0).astype(jnp.float32)
        inv_t[pl.ds(off, L)] = 1.0 / jnp.maximum(ct, 1.0)
        n0_t[pl.ds(off, L)] = float(SEQ) - ct
        inv_g[pl.ds(off, L)] = 1.0 / jnp.maximum(cg, 1.0)
        n0_g[pl.ds(off, L)] = float(SEQ) - cg
        return 0

    lax.fori_loop(0, BPW // L, cnt_body, 0)

    for c in range(NLCH):
        tcps[c].wait()
        pltpu.sync_copy(trows.at[c],
                        out_hbm.at[pl.ds(base + c * LCH, LCH), pl.ds(0, 32)])

    def fire_text(c, b):
        off = pl.multiple_of(c * TCH, TCH)
        for p in range(SEQ):
            pltpu.async_copy(
                xtab_hbm.at[ttok_v.at[p, pl.ds(off, TCH)]],
                tbuf.at[b, p], gsem.at[b])

    def drain_text(c, b):
        for p in range(SEQ):
            pltpu.make_async_copy(
                xtab_hbm.at[ttok_v.at[p, pl.ds(0, TCH)]],
                tbuf.at[b, p], gsem.at[b]).wait()

    def text_compute(c, b):
        off = pl.multiple_of(c * TCH, TCH)

        def grp_body(g, _):
            goff = pl.multiple_of(g * L, L)
            iv_vec = inv_t[pl.ds(pl.multiple_of(off + goff, L), L)]
            nv_vec = n0_t[pl.ds(pl.multiple_of(off + goff, L), L)]
            for j in range(L):
                r = goff + j
                iv = iv_vec[j]
                nv = nv_vec[j]
                for h in range(2):
                    s = tbuf[b, 0, r, pl.ds(h * L, L)]
                    for p in range(1, SEQ):
                        s = s + tbuf[b, p, r, pl.ds(h * L, L)]
                    t0 = ttab0[pl.ds(h * L, L)]
                    res_t[b, r, pl.ds(h * L, L)] = (s - nv * t0) * iv
            return 0

        lax.fori_loop(0, TCH // L, grp_body, 0)
        pltpu.async_copy(res_t.at[b],
                         out_hbm.at[pl.ds(base + off, TCH), pl.ds(32, 32)],
                         wsem.at[b])

    def drain_res_t(b):
        pltpu.make_async_copy(res_t.at[b],
                              out_hbm.at[pl.ds(0, TCH), pl.ds(32, 32)],
                              wsem.at[b]).wait()

    def fire_genre(c, b):
        off = pl.multiple_of(c * TCH, TCH)
        for p in range(SEQ):
            pltpu.async_copy(
                gtab_hbm.at[gtok_v.at[p, pl.ds(off, TCH)]],
                gbuf.at[b, p], hsem.at[b])

    def drain_genre(c, b):
        for p in range(SEQ):
            pltpu.make_async_copy(
                gtab_hbm.at[gtok_v.at[p, pl.ds(0, TCH)]],
                gbuf.at[b, p], hsem.at[b]).wait()

    def genre_compute(c, b):
        off = pl.multiple_of(c * TCH, TCH)

        def grp_body(g, _):
            goff = pl.multiple_of(g * L, L)
            iv_vec = inv_g[pl.ds(pl.multiple_of(off + goff, L), L)]
            nv_vec = n0_g[pl.ds(pl.multiple_of(off + goff, L), L)]
            for j in range(L):
                r = goff + j
                iv = iv_vec[j]
                nv = nv_vec[j]
                s = gbuf[b, 0, r, :]
                for p in range(1, SEQ):
                    s = s + gbuf[b, p, r, :]
                res_g[b, r, :] = (s - nv * gtab0[:]) * iv
            return 0

        lax.fori_loop(0, TCH // L, grp_body, 0)
        pltpu.async_copy(res_g.at[b],
                         out_hbm.at[pl.ds(base + off, TCH), pl.ds(64, 16)],
                         vsem.at[b])

    def drain_res_g(b):
        pltpu.make_async_copy(res_g.at[b],
                              out_hbm.at[pl.ds(0, TCH), pl.ds(64, 16)],
                              vsem.at[b]).wait()

    fire_text(0, 0)
    fire_text(1, 1)

    def text_pair(c0, _):
        for b in range(2):
            c = c0 + b
            drain_text(c, b)

            @pl.when(c >= 2)
            def _():
                drain_res_t(b)

            text_compute(c, b)

            @pl.when(c + 2 < NTCH)
            def _():
                fire_text(c + 2, b)
        return 0

    lax.fori_loop(0, NTCH // 2, lambda i, _: text_pair(i * 2, _), 0)
    drain_res_t(0)
    drain_res_t(1)

    fire_genre(0, 0)
    fire_genre(1, 1)

    def genre_pair(c0, _):
        for b in range(2):
            c = c0 + b
            drain_genre(c, b)

            @pl.when(c >= 2)
            def _():
                drain_res_g(b)

            genre_compute(c, b)

            @pl.when(c + 2 < NTCH)
            def _():
                fire_genre(c + 2, b)
        return 0

    lax.fori_loop(0, NTCH // 2, lambda i, _: genre_pair(i * 2, _), 0)
    drain_res_g(0)
    drain_res_g(1)


@jax.jit
def _run(tidx, ttok_t, gtok_t, ttab, xtab, gtab):
    mesh = plsc.VectorSubcoreMesh(
        core_axis_name="c", subcore_axis_name="s",
        num_cores=NC, num_subcores=NS)
    return pl.kernel(
        _body,
        out_type=jax.ShapeDtypeStruct((B, 80), jnp.float32),
        mesh=mesh,
        scratch_types=[
            pltpu.VMEM((BPW,), jnp.int32),
            pltpu.VMEM((SEQ, BPW), jnp.int32),
            pltpu.VMEM((SEQ, BPW), jnp.int32),
            pltpu.VMEM((2, SEQ, TCH, 32), jnp.float32),
            pltpu.VMEM((2, SEQ, TCH, 16), jnp.float32),
            pltpu.VMEM((NLCH, LCH, 32), jnp.float32),
            pltpu.VMEM((2, TCH, 32), jnp.float32),
            pltpu.VMEM((2, TCH, 16), jnp.float32),
            pltpu.VMEM((BPW,), jnp.float32),
            pltpu.VMEM((BPW,), jnp.float32),
            pltpu.VMEM((BPW,), jnp.float32),
            pltpu.VMEM((BPW,), jnp.float32),
            pltpu.VMEM((32,), jnp.float32),
            pltpu.VMEM((16,), jnp.float32),
            pltpu.SemaphoreType.DMA,
            pltpu.SemaphoreType.DMA((2,)),
            pltpu.SemaphoreType.DMA((2,)),
            pltpu.SemaphoreType.DMA((2,)),
            pltpu.SemaphoreType.DMA((2,)),
        ],
        compiler_params=pltpu.CompilerParams(use_tc_tiling_on_sc=False),
    )(tidx, ttok_t, gtok_t, ttab, xtab, gtab)


def kernel(movie_title_idx, title_tokens, genre_tokens,
           title_table, text_table, genre_table):
    tidx = movie_title_idx.astype(jnp.int32)
    ttok_t = title_tokens.astype(jnp.int32).T
    gtok_t = genre_tokens.astype(jnp.int32).T
    return _run(tidx, ttok_t, gtok_t, title_table, text_table, genre_table)


# speedup vs baseline: 1.7638x; 1.0434x over previous
import jax
import jax.numpy as jnp
from jax import lax
from jax.experimental import pallas as pl
from jax.experimental.pallas import tpu as pltpu
from jax.experimental.pallas import tpu_sc as plsc

B = 16384
SEQ = 20
NC = 2
NS = 16
NW = NC * NS
BPW = B // NW
TCH = 32
NTCH = BPW // TCH
LCH = 64
NLCH = BPW // LCH
L = 16
RPS = BPW // 128


def _body(tidx_hbm, ttok_hbm, gtok_hbm, ttab_hbm, xtab_hbm, gtab_hbm,
          out_hbm,
          tidx_v, ttok_v, gtok_v, tbuf, gbuf, t128, res_t, res_g,
          inv_t, n0_t, inv_g, n0_g, ttab0, gtab0,
          tsem, gsem, hsem, wsem, vsem):
    wid = lax.axis_index("s") * NC + lax.axis_index("c")
    base = pl.multiple_of(wid * BPW, BPW)

    pltpu.sync_copy(tidx_hbm.at[pl.ds(base, BPW)], tidx_v)
    pltpu.sync_copy(ttok_hbm.at[:, pl.ds(wid * RPS, RPS)], ttok_v)
    pltpu.sync_copy(gtok_hbm.at[:, pl.ds(wid * RPS, RPS)], gtok_v)
    pltpu.sync_copy(xtab_hbm.at[0], ttab0)
    pltpu.sync_copy(gtab_hbm.at[0], gtab0)

    def tfire(c, b):
        off = pl.multiple_of(c * LCH, LCH)
        pltpu.async_copy(
            ttab_hbm.at[tidx_v.at[pl.ds(off, LCH)]], t128.at[b], tsem.at[b])

    def tdrain(b):
        pltpu.make_async_copy(
            ttab_hbm.at[tidx_v.at[pl.ds(0, LCH)]], t128.at[b],
            tsem.at[b]).wait()

    tfire(0, 0)
    tfire(1, 1)

    def cnt_body(i, _):
        off = pl.multiple_of(i * L, L)
        row = lax.div(i, 8)
        lo = pl.multiple_of(lax.rem(i, 8) * L, L)
        ct = jnp.zeros((L,), jnp.float32)
        cg = jnp.zeros((L,), jnp.float32)
        for p in range(SEQ):
            t = ttok_v[p, row, pl.ds(lo, L)]
            g = gtok_v[p, row, pl.ds(lo, L)]
            ct = ct + jnp.where(t != 0, 1.0, 0.0).astype(jnp.float32)
            cg = cg + jnp.where(g != 0, 1.0, 0.0).astype(jnp.float32)
        inv_t[pl.ds(off, L)] = 1.0 / jnp.maximum(ct, 1.0)
        n0_t[pl.ds(off, L)] = float(SEQ) - ct
        inv_g[pl.ds(off, L)] = 1.0 / jnp.maximum(cg, 1.0)
        n0_g[pl.ds(off, L)] = float(SEQ) - cg
        return 0

    lax.fori_loop(0, BPW // L, cnt_body, 0)

    for c in range(NLCH):
        b = c % 2
        tdrain(b)
        pltpu.sync_copy(t128.at[b],
                        out_hbm.at[pl.ds(base + c * LCH, LCH)])
        if c + 2 < NLCH:
            tfire(c + 2, b)

    def tok_slice(tok_ref, p, c):
        row = lax.div(c, 4)
        lo = pl.multiple_of(lax.rem(c, 4) * TCH, TCH)
        return tok_ref.at[p, row, pl.ds(lo, TCH)]

    def fire_text(c, b):
        for p in range(SEQ):
            pltpu.async_copy(
                xtab_hbm.at[tok_slice(ttok_v, p, c)],
                tbuf.at[b, p], gsem.at[b])

    def drain_text(b):
        for p in range(SEQ):
            pltpu.make_async_copy(
                xtab_hbm.at[ttok_v.at[p, 0, pl.ds(0, TCH)]],
                tbuf.at[b, p], gsem.at[b]).wait()

    def text_compute(c, b):
        off = pl.multiple_of(c * TCH, TCH)

        def grp_body(g, _):
            goff = pl.multiple_of(g * L, L)
            iv_vec = inv_t[pl.ds(pl.multiple_of(off + goff, L), L)]
            nv_vec = n0_t[pl.ds(pl.multiple_of(off + goff, L), L)]
            for j in range(L):
                r = goff + j
                iv = iv_vec[j]
                nv = nv_vec[j]
                for h in range(2):
                    s = tbuf[b, 0, r, pl.ds(h * L, L)]
                    for p in range(1, SEQ):
                        s = s + tbuf[b, p, r, pl.ds(h * L, L)]
                    t0 = ttab0[pl.ds(h * L, L)]
                    res_t[b, r, pl.ds(h * L, L)] = (s - nv * t0) * iv
            return 0

        lax.fori_loop(0, TCH // L, grp_body, 0)
        pltpu.async_copy(res_t.at[b],
                         out_hbm.at[pl.ds(base + off, TCH), pl.ds(32, 32)],
                         wsem.at[b])

    def drain_res_t(b):
        pltpu.make_async_copy(res_t.at[b],
                              out_hbm.at[pl.ds(0, TCH), pl.ds(32, 32)],
                              wsem.at[b]).wait()

    def fire_genre(c, b):
        for p in range(SEQ):
            pltpu.async_copy(
                gtab_hbm.at[tok_slice(gtok_v, p, c)],
                gbuf.at[b, p], hsem.at[b])

    def drain_genre(b):
        for p in range(SEQ):
            pltpu.make_async_copy(
                gtab_hbm.at[gtok_v.at[p, 0, pl.ds(0, TCH)]],
                gbuf.at[b, p], hsem.at[b]).wait()

    def genre_compute(c, b):
        off = pl.multiple_of(c * TCH, TCH)

        def grp_body(g, _):
            goff = pl.multiple_of(g * L, L)
            iv_vec = inv_g[pl.ds(pl.multiple_of(off + goff, L), L)]
            nv_vec = n0_g[pl.ds(pl.multiple_of(off + goff, L), L)]
            for j in range(L):
                r = goff + j
                iv = iv_vec[j]
                nv = nv_vec[j]
                s = gbuf[b, 0, r, :]
                for p in range(1, SEQ):
                    s = s + gbuf[b, p, r, :]
                res_g[b, r, :] = (s - nv * gtab0[:]) * iv
            return 0

        lax.fori_loop(0, TCH // L, grp_body, 0)
        pltpu.async_copy(res_g.at[b],
                         out_hbm.at[pl.ds(base + off, TCH), pl.ds(64, 16)],
                         vsem.at[b])

    def drain_res_g(b):
        pltpu.make_async_copy(res_g.at[b],
                              out_hbm.at[pl.ds(0, TCH), pl.ds(64, 16)],
                              vsem.at[b]).wait()

    fire_text(0, 0)
    fire_text(1, 1)

    def text_pair(c0, _):
        for b in range(2):
            c = c0 + b
            drain_text(b)

            @pl.when(c >= 2)
            def _():
                drain_res_t(b)

            text_compute(c, b)

            @pl.when(c + 2 < NTCH)
            def _():
                fire_text(c + 2, b)
        return 0

    lax.fori_loop(0, NTCH // 2, lambda i, _: text_pair(i * 2, _), 0)
    drain_res_t(0)
    drain_res_t(1)

    fire_genre(0, 0)
    fire_genre(1, 1)

    def genre_pair(c0, _):
        for b in range(2):
            c = c0 + b
            drain_genre(b)

            @pl.when(c >= 2)
            def _():
                drain_res_g(b)

            genre_compute(c, b)

            @pl.when(c + 2 < NTCH)
            def _():
                fire_genre(c + 2, b)
        return 0

    lax.fori_loop(0, NTCH // 2, lambda i, _: genre_pair(i * 2, _), 0)
    drain_res_g(0)
    drain_res_g(1)


@jax.jit
def _run(tidx, ttok3, gtok3, ttab128, xtab, gtab):
    mesh = plsc.VectorSubcoreMesh(
        core_axis_name="c", subcore_axis_name="s",
        num_cores=NC, num_subcores=NS)
    return pl.kernel(
        _body,
        out_type=jax.ShapeDtypeStruct((B, 128), jnp.float32),
        mesh=mesh,
        scratch_types=[
            pltpu.VMEM((BPW,), jnp.int32),
            pltpu.VMEM((SEQ, RPS, 128), jnp.int32),
            pltpu.VMEM((SEQ, RPS, 128), jnp.int32),
            pltpu.VMEM((2, SEQ, TCH, 32), jnp.float32),
            pltpu.VMEM((2, SEQ, TCH, 16), jnp.float32),
            pltpu.VMEM((2, LCH, 128), jnp.float32),
            pltpu.VMEM((2, TCH, 32), jnp.float32),
            pltpu.VMEM((2, TCH, 16), jnp.float32),
            pltpu.VMEM((BPW,), jnp.float32),
            pltpu.VMEM((BPW,), jnp.float32),
            pltpu.VMEM((BPW,), jnp.float32),
            pltpu.VMEM((BPW,), jnp.float32),
            pltpu.VMEM((32,), jnp.float32),
            pltpu.VMEM((16,), jnp.float32),
            pltpu.SemaphoreType.DMA((2,)),
            pltpu.SemaphoreType.DMA((2,)),
            pltpu.SemaphoreType.DMA((2,)),
            pltpu.SemaphoreType.DMA((2,)),
            pltpu.SemaphoreType.DMA((2,)),
        ],
        compiler_params=pltpu.CompilerParams(use_tc_tiling_on_sc=False),
    )(tidx, ttok3, gtok3, ttab128, xtab, gtab)


def kernel(movie_title_idx, title_tokens, genre_tokens,
           title_table, text_table, genre_table):
    tidx = movie_title_idx.astype(jnp.int32)
    ttok3 = title_tokens.astype(jnp.int32).T.reshape(SEQ, B // 128, 128)
    gtok3 = genre_tokens.astype(jnp.int32).T.reshape(SEQ, B // 128, 128)
    ttab128 = jnp.pad(title_table, ((0, 7), (0, 96)))
    out = _run(tidx, ttok3, gtok3, ttab128, text_table, genre_table)
    return out[:, :80]


# speedup vs baseline: 1.8201x; 1.0319x over previous
import jax
import jax.numpy as jnp
from jax import lax
from jax.experimental import pallas as pl
from jax.experimental.pallas import tpu as pltpu
from jax.experimental.pallas import tpu_sc as plsc

B = 16384
SEQ = 20
NC = 2
NS = 16
NW = NC * NS
BPW = B // NW
TCH = 32
NTCH = BPW // TCH
LCH = 64
NLCH = BPW // LCH
L = 16
RPS = BPW // 128


def _body_pool(ttok_hbm, gtok_hbm, xtab_hbm, gtab_hbm,
               out_hbm,
               ttok_v, gtok_v, tbuf, gbuf, res_t, res_g,
               inv_t, n0_t, inv_g, n0_g, ttab0, gtab0,
               gsem, hsem, wsem, vsem):
    wid = lax.axis_index("s") * NC + lax.axis_index("c")
    base = pl.multiple_of(wid * BPW, BPW)

    pltpu.sync_copy(ttok_hbm.at[:, pl.ds(wid * RPS, RPS)], ttok_v)
    pltpu.sync_copy(gtok_hbm.at[:, pl.ds(wid * RPS, RPS)], gtok_v)
    pltpu.sync_copy(xtab_hbm.at[0], ttab0)
    pltpu.sync_copy(gtab_hbm.at[0], gtab0)

    def cnt_body(i, _):
        off = pl.multiple_of(i * L, L)
        row = lax.div(i, 8)
        lo = pl.multiple_of(lax.rem(i, 8) * L, L)
        ct = jnp.zeros((L,), jnp.float32)
        cg = jnp.zeros((L,), jnp.float32)
        for p in range(SEQ):
            t = ttok_v[p, row, pl.ds(lo, L)]
            g = gtok_v[p, row, pl.ds(lo, L)]
            ct = ct + jnp.where(t != 0, 1.0, 0.0).astype(jnp.float32)
            cg = cg + jnp.where(g != 0, 1.0, 0.0).astype(jnp.float32)
        inv_t[pl.ds(off, L)] = 1.0 / jnp.maximum(ct, 1.0)
        n0_t[pl.ds(off, L)] = float(SEQ) - ct
        inv_g[pl.ds(off, L)] = 1.0 / jnp.maximum(cg, 1.0)
        n0_g[pl.ds(off, L)] = float(SEQ) - cg
        return 0

    lax.fori_loop(0, BPW // L, cnt_body, 0)

    def tok_slice(tok_ref, p, c):
        row = lax.div(c, 4)
        lo = pl.multiple_of(lax.rem(c, 4) * TCH, TCH)
        return tok_ref.at[p, row, pl.ds(lo, TCH)]

    def fire_text(c, b):
        for p in range(SEQ):
            pltpu.async_copy(
                xtab_hbm.at[tok_slice(ttok_v, p, c)],
                tbuf.at[b, p], gsem.at[b])

    def drain_text(b):
        for p in range(SEQ):
            pltpu.make_async_copy(
                xtab_hbm.at[ttok_v.at[p, 0, pl.ds(0, TCH)]],
                tbuf.at[b, p], gsem.at[b]).wait()

    def text_compute(c, b):
        off = pl.multiple_of(c * TCH, TCH)

        def grp_body(g, _):
            goff = pl.multiple_of(g * L, L)
            iv_vec = inv_t[pl.ds(pl.multiple_of(off + goff, L), L)]
            nv_vec = n0_t[pl.ds(pl.multiple_of(off + goff, L), L)]
            for j in range(L):
                r = goff + j
                iv = iv_vec[j]
                nv = nv_vec[j]
                for h in range(2):
                    s = tbuf[b, 0, r, pl.ds(h * L, L)]
                    for p in range(1, SEQ):
                        s = s + tbuf[b, p, r, pl.ds(h * L, L)]
                    t0 = ttab0[pl.ds(h * L, L)]
                    res_t[b, r, pl.ds(h * L, L)] = (s - nv * t0) * iv
            return 0

        lax.fori_loop(0, TCH // L, grp_body, 0)
        pltpu.async_copy(res_t.at[b],
                         out_hbm.at[pl.ds(base + off, TCH), pl.ds(32, 32)],
                         wsem.at[b])

    def drain_res_t(b):
        pltpu.make_async_copy(res_t.at[b],
                              out_hbm.at[pl.ds(0, TCH), pl.ds(32, 32)],
                              wsem.at[b]).wait()

    def fire_genre(c, b):
        for p in range(SEQ):
            pltpu.async_copy(
                gtab_hbm.at[tok_slice(gtok_v, p, c)],
                gbuf.at[b, p], hsem.at[b])

    def drain_genre(b):
        for p in range(SEQ):
            pltpu.make_async_copy(
                gtab_hbm.at[gtok_v.at[p, 0, pl.ds(0, TCH)]],
                gbuf.at[b, p], hsem.at[b]).wait()

    def genre_compute(c, b):
        off = pl.multiple_of(c * TCH, TCH)

        def grp_body(g, _):
            goff = pl.multiple_of(g * L, L)
            iv_vec = inv_g[pl.ds(pl.multiple_of(off + goff, L), L)]
            nv_vec = n0_g[pl.ds(pl.multiple_of(off + goff, L), L)]
            for j in range(L):
                r = goff + j
                iv = iv_vec[j]
                nv = nv_vec[j]
                s = gbuf[b, 0, r, :]
                for p in range(1, SEQ):
                    s = s + gbuf[b, p, r, :]
                res_g[b, r, :] = (s - nv * gtab0[:]) * iv
            return 0

        lax.fori_loop(0, TCH // L, grp_body, 0)
        pltpu.async_copy(res_g.at[b],
                         out_hbm.at[pl.ds(base + off, TCH), pl.ds(64, 16)],
                         vsem.at[b])

    def drain_res_g(b):
        pltpu.make_async_copy(res_g.at[b],
                              out_hbm.at[pl.ds(0, TCH), pl.ds(64, 16)],
                              vsem.at[b]).wait()

    fire_text(0, 0)
    fire_text(1, 1)

    def text_pair(c0, _):
        for b in range(2):
            c = c0 + b
            drain_text(b)

            @pl.when(c >= 2)
            def _():
                drain_res_t(b)

            text_compute(c, b)

            @pl.when(c + 2 < NTCH)
            def _():
                fire_text(c + 2, b)
        return 0

    lax.fori_loop(0, NTCH // 2, lambda i, _: text_pair(i * 2, _), 0)
    drain_res_t(0)
    drain_res_t(1)

    fire_genre(0, 0)
    fire_genre(1, 1)

    def genre_pair(c0, _):
        for b in range(2):
            c = c0 + b
            drain_genre(b)

            @pl.when(c >= 2)
            def _():
                drain_res_g(b)

            genre_compute(c, b)

            @pl.when(c + 2 < NTCH)
            def _():
                fire_genre(c + 2, b)
        return 0

    lax.fori_loop(0, NTCH // 2, lambda i, _: genre_pair(i * 2, _), 0)
    drain_res_g(0)
    drain_res_g(1)


def _body_title(tidx_hbm, ttab_hbm, pooled_hbm, out_hbm,
                tidx_v, t128, a128, tsem, rsem, wsem):
    wid = lax.axis_index("s") * NC + lax.axis_index("c")
    base = pl.multiple_of(wid * BPW, BPW)

    pltpu.sync_copy(tidx_hbm.at[pl.ds(base, BPW)], tidx_v)

    def fire(c):
        off = pl.multiple_of(c * LCH, LCH)
        pltpu.async_copy(
            ttab_hbm.at[tidx_v.at[pl.ds(off, LCH)]], t128.at[c % 2],
            tsem.at[c % 2])
        pltpu.async_copy(
            pooled_hbm.at[pl.ds(base + off, LCH)], a128.at[c % 4],
            rsem.at[c % 4])

    def drain(c):
        pltpu.make_async_copy(
            ttab_hbm.at[tidx_v.at[pl.ds(0, LCH)]], t128.at[c % 2],
            tsem.at[c % 2]).wait()
        pltpu.make_async_copy(
            pooled_hbm.at[pl.ds(0, LCH)], a128.at[c % 4],
            rsem.at[c % 4]).wait()

    def drain_w(c):
        pltpu.make_async_copy(a128.at[c % 4], out_hbm.at[pl.ds(0, LCH)],
                              wsem.at[c % 4]).wait()

    fire(0)
    fire(1)
    for c in range(NLCH):
        drain(c)
        if c >= 2:
            drain_w(c - 2)

        def merge_body(r, _):
            a128[c % 4, r, pl.ds(0, L)] = t128[c % 2, r, pl.ds(0, L)]
            a128[c % 4, r, pl.ds(L, L)] = t128[c % 2, r, pl.ds(L, L)]
            return 0

        lax.fori_loop(0, LCH, merge_body, 0)
        pltpu.async_copy(a128.at[c % 4],
                         out_hbm.at[pl.ds(base + c * LCH, LCH)],
                         wsem.at[c % 4])
        if c + 2 < NLCH:
            fire(c + 2)
    drain_w(NLCH - 2)
    drain_w(NLCH - 1)


@jax.jit
def _run(tidx, ttok3, gtok3, ttab128, xtab, gtab):
    mesh = plsc.VectorSubcoreMesh(
        core_axis_name="c", subcore_axis_name="s",
        num_cores=NC, num_subcores=NS)
    pooled = pl.kernel(
        _body_pool,
        out_type=jax.ShapeDtypeStruct((B, 128), jnp.float32),
        mesh=mesh,
        scratch_types=[
            pltpu.VMEM((SEQ, RPS, 128), jnp.int32),
            pltpu.VMEM((SEQ, RPS, 128), jnp.int32),
            pltpu.VMEM((2, SEQ, TCH, 32), jnp.float32),
            pltpu.VMEM((2, SEQ, TCH, 16), jnp.float32),
            pltpu.VMEM((2, TCH, 32), jnp.float32),
            pltpu.VMEM((2, TCH, 16), jnp.float32),
            pltpu.VMEM((BPW,), jnp.float32),
            pltpu.VMEM((BPW,), jnp.float32),
            pltpu.VMEM((BPW,), jnp.float32),
            pltpu.VMEM((BPW,), jnp.float32),
            pltpu.VMEM((32,), jnp.float32),
            pltpu.VMEM((16,), jnp.float32),
            pltpu.SemaphoreType.DMA((2,)),
            pltpu.SemaphoreType.DMA((2,)),
            pltpu.SemaphoreType.DMA((2,)),
            pltpu.SemaphoreType.DMA((2,)),
        ],
        compiler_params=pltpu.CompilerParams(use_tc_tiling_on_sc=False),
    )(ttok3, gtok3, xtab, gtab)
    return pl.kernel(
        _body_title,
        out_type=jax.ShapeDtypeStruct((B, 128), jnp.float32),
        mesh=mesh,
        scratch_types=[
            pltpu.VMEM((BPW,), jnp.int32),
            pltpu.VMEM((2, LCH, 128), jnp.float32),
            pltpu.VMEM((4, LCH, 128), jnp.float32),
            pltpu.SemaphoreType.DMA((2,)),
            pltpu.SemaphoreType.DMA((4,)),
            pltpu.SemaphoreType.DMA((4,)),
        ],
        compiler_params=pltpu.CompilerParams(use_tc_tiling_on_sc=False),
    )(tidx, ttab128, pooled)


def kernel(movie_title_idx, title_tokens, genre_tokens,
           title_table, text_table, genre_table):
    tidx = movie_title_idx.astype(jnp.int32)
    ttok3 = title_tokens.astype(jnp.int32).T.reshape(SEQ, B // 128, 128)
    gtok3 = genre_tokens.astype(jnp.int32).T.reshape(SEQ, B // 128, 128)
    ttab128 = jnp.pad(title_table, ((0, 7), (0, 96)))
    out = _run(tidx, ttok3, gtok3, ttab128, text_table, genre_table)
    return out[:, :80]
